# baseline (device time: 203403 ns/iter reference)
import jax
import jax.numpy as jnp
from jax import lax
from jax.experimental import pallas as pl
from jax.experimental.pallas import tpu as pltpu

N_DEV = 8


def kernel(x, W1, W2):
    m, _ = x.shape
    n = W2.shape[1]
    ch = m // N_DEV

    def body(x_ref, w1_ref, w2_ref, out_ref, tx_ref, rx_ref,
             send_sem, recv_sem, credit_sem):
        my = lax.axis_index("i")
        left = lax.rem(my - 1 + N_DEV, N_DEV)
        right = lax.rem(my + 1, N_DEV)

        barrier_sem = pltpu.get_barrier_semaphore()
        for nbr in (left, right):
            pl.semaphore_signal(barrier_sem, inc=1, device_id=(nbr,),
                                device_id_type=pl.DeviceIdType.MESH)
        pl.semaphore_wait(barrier_sem, 2)

        h = jnp.dot(x_ref[:, :], w1_ref[:, :],
                    preferred_element_type=jnp.float32)
        h = jnp.maximum(h, 0.0).astype(jnp.bfloat16)
        out_ref[:, :] = jnp.dot(h, w2_ref[:, :],
                                preferred_element_type=jnp.float32)

        def chunk(idx):
            return pl.ds(idx * ch, ch)

        for s in range(2 * (N_DEV - 1)):
            if s < N_DEV - 1:
                c_send = lax.rem(my - s + 2 * N_DEV, N_DEV)
            else:
                t = s - (N_DEV - 1)
                c_send = lax.rem(my + 1 - t + 2 * N_DEV, N_DEV)
            tx_ref[:, :] = out_ref[chunk(c_send), :].astype(jnp.bfloat16)
            if s > 0:
                pl.semaphore_wait(credit_sem, 1)
            rdma = pltpu.make_async_remote_copy(
                src_ref=tx_ref, dst_ref=rx_ref,
                send_sem=send_sem, recv_sem=recv_sem,
                device_id=(right,), device_id_type=pl.DeviceIdType.MESH)
            rdma.start()
            rdma.wait()
            if s < N_DEV - 1:
                c_recv = lax.rem(my - s - 1 + 2 * N_DEV, N_DEV)
                out_ref[chunk(c_recv), :] = (
                    out_ref[chunk(c_recv), :] + rx_ref[:, :].astype(jnp.float32))
            else:
                t = s - (N_DEV - 1)
                c_recv = lax.rem(my - t + 2 * N_DEV, N_DEV)
                out_ref[chunk(c_recv), :] = rx_ref[:, :].astype(jnp.float32)
            if s < 2 * (N_DEV - 1) - 1:
                pl.semaphore_signal(credit_sem, inc=1, device_id=(left,),
                                    device_id_type=pl.DeviceIdType.MESH)

    return pl.pallas_call(
        body,
        out_shape=jax.ShapeDtypeStruct((m, n), jnp.float32),
        in_specs=[pl.BlockSpec(memory_space=pltpu.VMEM)] * 3,
        out_specs=pl.BlockSpec(memory_space=pltpu.VMEM),
        scratch_shapes=[
            pltpu.VMEM((ch, n), jnp.bfloat16),
            pltpu.VMEM((ch, n), jnp.bfloat16),
            pltpu.SemaphoreType.DMA,
            pltpu.SemaphoreType.DMA,
            pltpu.SemaphoreType.REGULAR,
        ],
        compiler_params=pltpu.CompilerParams(collective_id=0),
    )(x.astype(jnp.bfloat16), W1.astype(jnp.bfloat16), W2.astype(jnp.bfloat16))


# device time: 176461 ns/iter; 1.1527x vs baseline; 1.1527x over previous
import jax
import jax.numpy as jnp
from jax import lax
from jax.experimental import pallas as pl
from jax.experimental.pallas import tpu as pltpu

N_DEV = 8


def kernel(x, W1, W2):
    m, _ = x.shape
    n = W2.shape[1]
    ch = m // N_DEV

    def body(x_ref, w1_ref, w2_ref, out_ref, tx_ref, rx_ref,
             send_sem, recv_sem, credit_sem):
        my = lax.axis_index("i")
        left = lax.rem(my - 1 + N_DEV, N_DEV)
        right = lax.rem(my + 1, N_DEV)

        barrier_sem = pltpu.get_barrier_semaphore()
        for nbr in (left, right):
            pl.semaphore_signal(barrier_sem, inc=1, device_id=(nbr,),
                                device_id_type=pl.DeviceIdType.MESH)
        pl.semaphore_wait(barrier_sem, 2)

        def chunk(idx):
            return pl.ds(idx * ch, ch)

        def compute_chunk(c):
            rows = chunk(c)
            hc = jnp.dot(x_ref[rows, :], w1_ref[:, :],
                         preferred_element_type=jnp.float32)
            hc = jnp.maximum(hc, 0.0).astype(jnp.bfloat16)
            out_ref[rows, :] = jnp.dot(hc, w2_ref[:, :],
                                       preferred_element_type=jnp.float32)

        compute_chunk(my)

        for s in range(2 * (N_DEV - 1)):
            if s < N_DEV - 1:
                c_send = lax.rem(my - s + 2 * N_DEV, N_DEV)
            else:
                t = s - (N_DEV - 1)
                c_send = lax.rem(my + 1 - t + 2 * N_DEV, N_DEV)
            tx_ref[:, :] = out_ref[chunk(c_send), :].astype(jnp.bfloat16)
            if s > 0:
                pl.semaphore_wait(credit_sem, 1)
            rdma = pltpu.make_async_remote_copy(
                src_ref=tx_ref, dst_ref=rx_ref,
                send_sem=send_sem, recv_sem=recv_sem,
                device_id=(right,), device_id_type=pl.DeviceIdType.MESH)
            rdma.start()
            if s < N_DEV - 1:
                c_recv = lax.rem(my - s - 1 + 2 * N_DEV, N_DEV)
                compute_chunk(c_recv)
                rdma.wait()
                out_ref[chunk(c_recv), :] = (
                    out_ref[chunk(c_recv), :] + rx_ref[:, :].astype(jnp.float32))
            else:
                t = s - (N_DEV - 1)
                c_recv = lax.rem(my - t + 2 * N_DEV, N_DEV)
                rdma.wait()
                out_ref[chunk(c_recv), :] = rx_ref[:, :].astype(jnp.float32)
            if s < 2 * (N_DEV - 1) - 1:
                pl.semaphore_signal(credit_sem, inc=1, device_id=(left,),
                                    device_id_type=pl.DeviceIdType.MESH)

    return pl.pallas_call(
        body,
        out_shape=jax.ShapeDtypeStruct((m, n), jnp.float32),
        in_specs=[pl.BlockSpec(memory_space=pltpu.VMEM)] * 3,
        out_specs=pl.BlockSpec(memory_space=pltpu.VMEM),
        scratch_shapes=[
            pltpu.VMEM((ch, n), jnp.bfloat16),
            pltpu.VMEM((ch, n), jnp.bfloat16),
            pltpu.SemaphoreType.DMA,
            pltpu.SemaphoreType.DMA,
            pltpu.SemaphoreType.REGULAR,
        ],
        compiler_params=pltpu.CompilerParams(collective_id=0),
    )(x.astype(jnp.bfloat16), W1.astype(jnp.bfloat16), W2.astype(jnp.bfloat16))


# device time: 147229 ns/iter; 1.3815x vs baseline; 1.1985x over previous
import jax
import jax.numpy as jnp
from jax import lax
from jax.experimental import pallas as pl
from jax.experimental.pallas import tpu as pltpu

N_DEV = 8


def kernel(x, W1, W2):
    m, _ = x.shape
    n = W2.shape[1]
    ch = m // N_DEV

    def body(x_ref, w1_ref, w2_ref, out_ref,
             tx_cw, rx_cw, tx_ccw, rx_ccw,
             send_cw, recv_cw, send_ccw, recv_ccw,
             credit_cw, credit_ccw):
        my = lax.axis_index("i")
        left = lax.rem(my - 1 + N_DEV, N_DEV)
        right = lax.rem(my + 1, N_DEV)

        def cidx(off):
            return lax.rem(my + off + 4 * N_DEV, N_DEV)

        def chunk(idx):
            return pl.ds(idx * ch, ch)

        barrier_sem = pltpu.get_barrier_semaphore()
        for nbr in (left, right):
            pl.semaphore_signal(barrier_sem, inc=1, device_id=(nbr,),
                                device_id_type=pl.DeviceIdType.MESH)
        pl.semaphore_wait(barrier_sem, 2)

        def compute_chunk(c):
            rows = chunk(c)
            hc = jnp.dot(x_ref[rows, :], w1_ref[:, :],
                         preferred_element_type=jnp.float32)
            hc = jnp.maximum(hc, 0.0).astype(jnp.bfloat16)
            out_ref[rows, :] = jnp.dot(hc, w2_ref[:, :],
                                       preferred_element_type=jnp.float32)

        def send_to(tx, rx, ssem, rsem, tgt):
            return pltpu.make_async_remote_copy(
                src_ref=tx, dst_ref=rx, send_sem=ssem, recv_sem=rsem,
                device_id=(tgt,), device_id_type=pl.DeviceIdType.MESH)

        compute_chunk(my)

        for s in range(N_DEV - 1):
            tx_cw[:, :] = out_ref[chunk(cidx(-s)), :].astype(jnp.bfloat16)
            if s > 0:
                pl.semaphore_wait(credit_cw, 1)
            rdma = send_to(tx_cw, rx_cw, send_cw, recv_cw, right)
            rdma.start()
            compute_chunk(cidx(-s - 1))
            rdma.wait()
            out_ref[chunk(cidx(-s - 1)), :] = (
                out_ref[chunk(cidx(-s - 1)), :]
                + rx_cw[:, :].astype(jnp.float32))
            pl.semaphore_signal(credit_cw, inc=1, device_id=(left,),
                                device_id_type=pl.DeviceIdType.MESH)


        for t in range(4):
            tx_cw[:, :] = out_ref[chunk(cidx(1 - t)), :].astype(jnp.bfloat16)
            pl.semaphore_wait(credit_cw, 1)
            rdma_cw = send_to(tx_cw, rx_cw, send_cw, recv_cw, right)
            rdma_cw.start()
            if t < 3:
                tx_ccw[:, :] = out_ref[chunk(cidx(1 + t)), :].astype(
                    jnp.bfloat16)
                if t > 0:
                    pl.semaphore_wait(credit_ccw, 1)
                rdma_ccw = send_to(tx_ccw, rx_ccw, send_ccw, recv_ccw, left)
                rdma_ccw.start()
            rdma_cw.wait()
            out_ref[chunk(cidx(-t)), :] = rx_cw[:, :].astype(jnp.float32)
            if t < 3:
                pl.semaphore_signal(credit_cw, inc=1, device_id=(left,),
                                    device_id_type=pl.DeviceIdType.MESH)
                rdma_ccw.wait()
                out_ref[chunk(cidx(2 + t)), :] = rx_ccw[:, :].astype(
                    jnp.float32)
                if t < 2:
                    pl.semaphore_signal(credit_ccw, inc=1,
                                        device_id=(right,),
                                        device_id_type=pl.DeviceIdType.MESH)

    return pl.pallas_call(
        body,
        out_shape=jax.ShapeDtypeStruct((m, n), jnp.float32),
        in_specs=[pl.BlockSpec(memory_space=pltpu.VMEM)] * 3,
        out_specs=pl.BlockSpec(memory_space=pltpu.VMEM),
        scratch_shapes=[
            pltpu.VMEM((ch, n), jnp.bfloat16),
            pltpu.VMEM((ch, n), jnp.bfloat16),
            pltpu.VMEM((ch, n), jnp.bfloat16),
            pltpu.VMEM((ch, n), jnp.bfloat16),
            pltpu.SemaphoreType.DMA,
            pltpu.SemaphoreType.DMA,
            pltpu.SemaphoreType.DMA,
            pltpu.SemaphoreType.DMA,
            pltpu.SemaphoreType.REGULAR,
            pltpu.SemaphoreType.REGULAR,
        ],
        compiler_params=pltpu.CompilerParams(collective_id=0),
    )(x.astype(jnp.bfloat16), W1.astype(jnp.bfloat16), W2.astype(jnp.bfloat16))
